# baseline (speedup 1.0000x reference)
import jax
import jax.numpy as jnp
from jax.experimental import pallas as pl
from jax.experimental.pallas import tpu as pltpu

_ROWS = 128
_COLS = 100000
_BR = 8
_N_STEPS = _ROWS // _BR
_CW = 12800
_NCHUNK = 7
_REMW = _COLS - _NCHUNK * _CW
_REMOFF = _NCHUNK * _CW

_ROTS = ((13, 15, 26, 6), (17, 29, 16, 24))


def _sample_block(p_block, base):
    R, W = p_block.shape
    row = jax.lax.broadcasted_iota(jnp.uint32, (R, W), 0)
    col = jax.lax.broadcasted_iota(jnp.uint32, (R, W), 1)
    idx = base.astype(jnp.uint32) + row * jnp.uint32(_COLS) + col

    k0 = jnp.uint32(0)
    k1 = jnp.uint32(42)
    ks = (k0, k1, k0 ^ k1 ^ jnp.uint32(0x1BD11BDA))
    x0 = jnp.full_like(idx, k0)
    x1 = idx + k1
    for i in range(5):
        for rot in _ROTS[i % 2]:
            x0 = x0 + x1
            x1 = (x1 << rot) | (x1 >> (32 - rot))
            x1 = x1 ^ x0
        x0 = x0 + ks[(i + 1) % 3]
        x1 = x1 + ks[(i + 2) % 3] + jnp.uint32(i + 1)

    bits = x0 ^ x1
    fb = (bits >> jnp.uint32(9)) | jnp.uint32(0x3F800000)
    u = jax.lax.bitcast_convert_type(fb, jnp.float32) - jnp.float32(1.0)
    return (u < p_block).astype(jnp.float32)


def _pipelined(p_hbm, o_hbm, p_vmem, o_vmem, p_rem, o_rem,
               in_sem, out_sem, rin_sem, rout_sem):
    def in_copy(step, slot, c):
        return pltpu.make_async_copy(
            p_hbm.at[pl.ds(step * _BR, _BR), pl.ds(c * _CW, _CW)],
            p_vmem.at[slot, c], in_sem.at[slot, c])

    def out_copy(step, slot, c):
        return pltpu.make_async_copy(
            o_vmem.at[slot, c],
            o_hbm.at[pl.ds(step * _BR, _BR), pl.ds(c * _CW, _CW)],
            out_sem.at[slot, c])

    def rin_copy(step, slot):
        return pltpu.make_async_copy(
            p_hbm.at[pl.ds(step * _BR, _BR), pl.ds(_REMOFF, _REMW)],
            p_rem.at[slot], rin_sem.at[slot])

    def rout_copy(step, slot):
        return pltpu.make_async_copy(
            o_rem.at[slot],
            o_hbm.at[pl.ds(step * _BR, _BR), pl.ds(_REMOFF, _REMW)],
            rout_sem.at[slot])

    def start_in(step, slot):
        for c in range(_NCHUNK):
            in_copy(step, slot, c).start()
        rin_copy(step, slot).start()

    start_in(0, 0)

    def body(step, carry):
        slot = jax.lax.rem(step, 2)

        @pl.when(step + 1 < _N_STEPS)
        def _():
            start_in(step + 1, 1 - slot)

        row_base = step * (_BR * _COLS)

        def chunk(c, ccarry):
            in_copy(step, slot, c).wait()

            @pl.when(step >= 2)
            def _():
                out_copy(step - 2, slot, c).wait()

            o_vmem[slot, c] = _sample_block(p_vmem[slot, c],
                                            row_base + c * _CW)
            out_copy(step, slot, c).start()
            return ccarry

        jax.lax.fori_loop(0, _NCHUNK, chunk, 0)

        rin_copy(step, slot).wait()

        @pl.when(step >= 2)
        def _():
            rout_copy(step - 2, slot).wait()

        o_rem[slot] = _sample_block(p_rem[slot], row_base + _REMOFF)
        rout_copy(step, slot).start()
        return carry

    jax.lax.fori_loop(0, _N_STEPS, body, 0)

    def final_wait(step, slot):
        for c in range(_NCHUNK):
            out_copy(step, slot, c).wait()
        rout_copy(step, slot).wait()

    final_wait(_N_STEPS - 2, 0)
    final_wait(_N_STEPS - 1, 1)


def kernel(probs):
    return pl.pallas_call(
        _pipelined,
        in_specs=[pl.BlockSpec(memory_space=pl.ANY)],
        out_specs=pl.BlockSpec(memory_space=pl.ANY),
        out_shape=jax.ShapeDtypeStruct((_ROWS, _COLS), probs.dtype),
        scratch_shapes=[
            pltpu.VMEM((2, _NCHUNK, _BR, _CW), jnp.float32),
            pltpu.VMEM((2, _NCHUNK, _BR, _CW), jnp.float32),
            pltpu.VMEM((2, _BR, _REMW), jnp.float32),
            pltpu.VMEM((2, _BR, _REMW), jnp.float32),
            pltpu.SemaphoreType.DMA((2, _NCHUNK)),
            pltpu.SemaphoreType.DMA((2, _NCHUNK)),
            pltpu.SemaphoreType.DMA((2,)),
            pltpu.SemaphoreType.DMA((2,)),
        ],
    )(probs)


# speedup vs baseline: 1.0127x; 1.0127x over previous
import jax
import jax.numpy as jnp
from jax.experimental import pallas as pl
from jax.experimental.pallas import tpu as pltpu

_ROWS = 128
_COLS = 100000
_BR = 8
_N_STEPS = _ROWS // _BR
_CW = 12800
_NCHUNK = 7
_REMW = _COLS - _NCHUNK * _CW
_REMOFF = _NCHUNK * _CW

_ROTS = ((13, 15, 26, 6), (17, 29, 16, 24))


def _sample_block(p_block, pattern, base):
    k0 = jnp.uint32(0)
    k1 = jnp.uint32(42)
    ks = (k0, k1, k0 ^ k1 ^ jnp.uint32(0x1BD11BDA))
    x1 = pattern + (base + 42).astype(jnp.uint32)
    x0 = x1
    x1 = ((x1 << 13) | (x1 >> 19)) ^ x0
    for i in range(5):
        for rot in _ROTS[i % 2][1 if i == 0 else 0:]:
            x0 = x0 + x1
            x1 = (x1 << rot) | (x1 >> (32 - rot))
            x1 = x1 ^ x0
        x0 = x0 + ks[(i + 1) % 3]
        x1 = x1 + ks[(i + 2) % 3] + jnp.uint32(i + 1)

    bits = x0 ^ x1
    fb = (bits >> jnp.uint32(9)) | jnp.uint32(0x3F800000)
    u = jax.lax.bitcast_convert_type(fb, jnp.float32) - jnp.float32(1.0)
    return (u < p_block).astype(jnp.float32)


def _pipelined(p_hbm, o_hbm, p_vmem, o_vmem, p_rem, o_rem, pat_ref,
               in_sem, out_sem, rin_sem, rout_sem):
    row = jax.lax.broadcasted_iota(jnp.uint32, (_BR, _CW), 0)
    col = jax.lax.broadcasted_iota(jnp.uint32, (_BR, _CW), 1)
    pat_ref[...] = row * jnp.uint32(_COLS) + col
    def in_copy(step, slot, c):
        return pltpu.make_async_copy(
            p_hbm.at[pl.ds(step * _BR, _BR), pl.ds(c * _CW, _CW)],
            p_vmem.at[slot, c], in_sem.at[slot, c])

    def out_copy(step, slot, c):
        return pltpu.make_async_copy(
            o_vmem.at[slot, c],
            o_hbm.at[pl.ds(step * _BR, _BR), pl.ds(c * _CW, _CW)],
            out_sem.at[slot, c])

    def rin_copy(step, slot):
        return pltpu.make_async_copy(
            p_hbm.at[pl.ds(step * _BR, _BR), pl.ds(_REMOFF, _REMW)],
            p_rem.at[slot], rin_sem.at[slot])

    def rout_copy(step, slot):
        return pltpu.make_async_copy(
            o_rem.at[slot],
            o_hbm.at[pl.ds(step * _BR, _BR), pl.ds(_REMOFF, _REMW)],
            rout_sem.at[slot])

    def start_in(step, slot):
        for c in range(_NCHUNK):
            in_copy(step, slot, c).start()
        rin_copy(step, slot).start()

    start_in(0, 0)

    def body(step, carry):
        slot = jax.lax.rem(step, 2)

        @pl.when(step + 1 < _N_STEPS)
        def _():
            start_in(step + 1, 1 - slot)

        row_base = step * (_BR * _COLS)

        def chunk(c, ccarry):
            in_copy(step, slot, c).wait()

            @pl.when(step >= 2)
            def _():
                out_copy(step - 2, slot, c).wait()

            o_vmem[slot, c] = _sample_block(p_vmem[slot, c], pat_ref[...],
                                            row_base + c * _CW)
            out_copy(step, slot, c).start()
            return ccarry

        jax.lax.fori_loop(0, _NCHUNK, chunk, 0)

        rin_copy(step, slot).wait()

        @pl.when(step >= 2)
        def _():
            rout_copy(step - 2, slot).wait()

        o_rem[slot] = _sample_block(p_rem[slot], pat_ref[:, :_REMW],
                                    row_base + _REMOFF)
        rout_copy(step, slot).start()
        return carry

    jax.lax.fori_loop(0, _N_STEPS, body, 0)

    def final_wait(step, slot):
        for c in range(_NCHUNK):
            out_copy(step, slot, c).wait()
        rout_copy(step, slot).wait()

    final_wait(_N_STEPS - 2, 0)
    final_wait(_N_STEPS - 1, 1)


def kernel(probs):
    return pl.pallas_call(
        _pipelined,
        in_specs=[pl.BlockSpec(memory_space=pl.ANY)],
        out_specs=pl.BlockSpec(memory_space=pl.ANY),
        out_shape=jax.ShapeDtypeStruct((_ROWS, _COLS), probs.dtype),
        scratch_shapes=[
            pltpu.VMEM((2, _NCHUNK, _BR, _CW), jnp.float32),
            pltpu.VMEM((2, _NCHUNK, _BR, _CW), jnp.float32),
            pltpu.VMEM((2, _BR, _REMW), jnp.float32),
            pltpu.VMEM((2, _BR, _REMW), jnp.float32),
            pltpu.VMEM((_BR, _CW), jnp.uint32),
            pltpu.SemaphoreType.DMA((2, _NCHUNK)),
            pltpu.SemaphoreType.DMA((2, _NCHUNK)),
            pltpu.SemaphoreType.DMA((2,)),
            pltpu.SemaphoreType.DMA((2,)),
        ],
    )(probs)


# speedup vs baseline: 1.0202x; 1.0075x over previous
import jax
import jax.numpy as jnp
from jax.experimental import pallas as pl
from jax.experimental.pallas import tpu as pltpu

_ROWS = 128
_COLS = 100000
_BR = 16
_N_STEPS = _ROWS // _BR
_CW = 12800
_NCHUNK = 7
_REMW = _COLS - _NCHUNK * _CW
_REMOFF = _NCHUNK * _CW

_ROTS = ((13, 15, 26, 6), (17, 29, 16, 24))


def _sample_block(p_block, pattern, base):
    k0 = jnp.uint32(0)
    k1 = jnp.uint32(42)
    ks = (k0, k1, k0 ^ k1 ^ jnp.uint32(0x1BD11BDA))
    x1 = pattern + (base + 42).astype(jnp.uint32)
    x0 = x1
    x1 = ((x1 << 13) | (x1 >> 19)) ^ x0
    for i in range(5):
        for rot in _ROTS[i % 2][1 if i == 0 else 0:]:
            x0 = x0 + x1
            x1 = (x1 << rot) | (x1 >> (32 - rot))
            x1 = x1 ^ x0
        x0 = x0 + ks[(i + 1) % 3]
        x1 = x1 + ks[(i + 2) % 3] + jnp.uint32(i + 1)

    bits = x0 ^ x1
    fb = (bits >> jnp.uint32(9)) | jnp.uint32(0x3F800000)
    u = jax.lax.bitcast_convert_type(fb, jnp.float32) - jnp.float32(1.0)
    return (u < p_block).astype(jnp.float32)


def _pipelined(p_hbm, o_hbm, p_vmem, o_vmem, p_rem, o_rem, pat_ref,
               in_sem, out_sem, rin_sem, rout_sem):
    row = jax.lax.broadcasted_iota(jnp.uint32, (_BR, _CW), 0)
    col = jax.lax.broadcasted_iota(jnp.uint32, (_BR, _CW), 1)
    pat_ref[...] = row * jnp.uint32(_COLS) + col
    def in_copy(step, slot, c):
        return pltpu.make_async_copy(
            p_hbm.at[pl.ds(step * _BR, _BR), pl.ds(c * _CW, _CW)],
            p_vmem.at[slot, c], in_sem.at[slot, c])

    def out_copy(step, slot, c):
        return pltpu.make_async_copy(
            o_vmem.at[slot, c],
            o_hbm.at[pl.ds(step * _BR, _BR), pl.ds(c * _CW, _CW)],
            out_sem.at[slot, c])

    def rin_copy(step, slot):
        return pltpu.make_async_copy(
            p_hbm.at[pl.ds(step * _BR, _BR), pl.ds(_REMOFF, _REMW)],
            p_rem.at[slot], rin_sem.at[slot])

    def rout_copy(step, slot):
        return pltpu.make_async_copy(
            o_rem.at[slot],
            o_hbm.at[pl.ds(step * _BR, _BR), pl.ds(_REMOFF, _REMW)],
            rout_sem.at[slot])

    def start_in(step, slot):
        for c in range(_NCHUNK):
            in_copy(step, slot, c).start()
        rin_copy(step, slot).start()

    start_in(0, 0)

    def body(step, carry):
        slot = jax.lax.rem(step, 2)

        @pl.when(step + 1 < _N_STEPS)
        def _():
            start_in(step + 1, 1 - slot)

        row_base = step * (_BR * _COLS)

        def chunk(c, ccarry):
            in_copy(step, slot, c).wait()

            @pl.when(step >= 2)
            def _():
                out_copy(step - 2, slot, c).wait()

            o_vmem[slot, c] = _sample_block(p_vmem[slot, c], pat_ref[...],
                                            row_base + c * _CW)
            out_copy(step, slot, c).start()
            return ccarry

        jax.lax.fori_loop(0, _NCHUNK, chunk, 0)

        rin_copy(step, slot).wait()

        @pl.when(step >= 2)
        def _():
            rout_copy(step - 2, slot).wait()

        o_rem[slot] = _sample_block(p_rem[slot], pat_ref[:, :_REMW],
                                    row_base + _REMOFF)
        rout_copy(step, slot).start()
        return carry

    jax.lax.fori_loop(0, _N_STEPS, body, 0)

    def final_wait(step, slot):
        for c in range(_NCHUNK):
            out_copy(step, slot, c).wait()
        rout_copy(step, slot).wait()

    final_wait(_N_STEPS - 2, 0)
    final_wait(_N_STEPS - 1, 1)


def kernel(probs):
    return pl.pallas_call(
        _pipelined,
        in_specs=[pl.BlockSpec(memory_space=pl.ANY)],
        out_specs=pl.BlockSpec(memory_space=pl.ANY),
        out_shape=jax.ShapeDtypeStruct((_ROWS, _COLS), probs.dtype),
        scratch_shapes=[
            pltpu.VMEM((2, _NCHUNK, _BR, _CW), jnp.float32),
            pltpu.VMEM((2, _NCHUNK, _BR, _CW), jnp.float32),
            pltpu.VMEM((2, _BR, _REMW), jnp.float32),
            pltpu.VMEM((2, _BR, _REMW), jnp.float32),
            pltpu.VMEM((_BR, _CW), jnp.uint32),
            pltpu.SemaphoreType.DMA((2, _NCHUNK)),
            pltpu.SemaphoreType.DMA((2, _NCHUNK)),
            pltpu.SemaphoreType.DMA((2,)),
            pltpu.SemaphoreType.DMA((2,)),
        ],
    )(probs)


# speedup vs baseline: 1.0232x; 1.0029x over previous
import jax
import jax.numpy as jnp
from jax.experimental import pallas as pl
from jax.experimental.pallas import tpu as pltpu

_ROWS = 128
_COLS = 100000
_BR = 16
_N_STEPS = _ROWS // _BR
_CW = 12800
_NCHUNK = 7
_REMW = _COLS - _NCHUNK * _CW
_REMOFF = _NCHUNK * _CW

_ROTS = ((13, 15, 26, 6), (17, 29, 16, 24))


def _sample_block(p_block, pattern, base):
    k0 = jnp.uint32(0)
    k1 = jnp.uint32(42)
    ks = (k0, k1, k0 ^ k1 ^ jnp.uint32(0x1BD11BDA))
    x1 = pattern + (base + 42).astype(jnp.uint32)
    x0 = x1
    x1 = ((x1 << 13) | (x1 >> 19)) ^ x0
    for i in range(5):
        for rot in _ROTS[i % 2][1 if i == 0 else 0:]:
            x0 = x0 + x1
            x1 = (x1 << rot) | (x1 >> (32 - rot))
            x1 = x1 ^ x0
        x0 = x0 + ks[(i + 1) % 3]
        x1 = x1 + ks[(i + 2) % 3] + jnp.uint32(i + 1)

    bits = x0 ^ x1
    fb = (bits >> jnp.uint32(9)) | jnp.uint32(0x3F800000)
    u = jax.lax.bitcast_convert_type(fb, jnp.float32) - jnp.float32(1.0)
    return (u < p_block).astype(jnp.float32)


def _pipelined(p_hbm, o_hbm, p_vmem, o_vmem, p_rem, o_rem, pat_ref,
               in_sem, out_sem, rin_sem, rout_sem):
    row = jax.lax.broadcasted_iota(jnp.uint32, (_BR, _CW), 0)
    col = jax.lax.broadcasted_iota(jnp.uint32, (_BR, _CW), 1)
    pat_ref[...] = row * jnp.uint32(_COLS) + col
    def in_copy(step, slot, c):
        return pltpu.make_async_copy(
            p_hbm.at[pl.ds(step * _BR, _BR), pl.ds(c * _CW, _CW)],
            p_vmem.at[slot, c], in_sem.at[slot, c])

    def out_copy(step, slot, c):
        return pltpu.make_async_copy(
            o_vmem.at[slot, c],
            o_hbm.at[pl.ds(step * _BR, _BR), pl.ds(c * _CW, _CW)],
            out_sem.at[slot, c])

    def rin_copy(step, slot):
        return pltpu.make_async_copy(
            p_hbm.at[pl.ds(step * _BR, _BR), pl.ds(_REMOFF, _REMW)],
            p_rem.at[slot], rin_sem.at[slot])

    def rout_copy(step, slot):
        return pltpu.make_async_copy(
            o_rem.at[slot],
            o_hbm.at[pl.ds(step * _BR, _BR), pl.ds(_REMOFF, _REMW)],
            rout_sem.at[slot])

    def start_in(step, slot):
        for c in range(_NCHUNK):
            in_copy(step, slot, c).start()
        rin_copy(step, slot).start()

    start_in(0, 0)

    def body(step, carry):
        slot = jax.lax.rem(step, 2)

        @pl.when(step + 1 < _N_STEPS)
        def _():
            start_in(step + 1, 1 - slot)

        row_base = step * (_BR * _COLS)

        def chunk(c, ccarry):
            in_copy(step, slot, c).wait()

            @pl.when(step >= 2)
            def _():
                out_copy(step - 2, slot, c).wait()

            o_vmem[slot, c] = _sample_block(p_vmem[slot, c], pat_ref[...],
                                            row_base + c * _CW)
            out_copy(step, slot, c).start()
            return ccarry

        jax.lax.fori_loop(0, _NCHUNK, chunk, 0)

        rin_copy(step, slot).wait()

        @pl.when(step >= 2)
        def _():
            rout_copy(step - 2, slot).wait()

        o_rem[slot] = _sample_block(p_rem[slot], pat_ref[:, :_REMW],
                                    row_base + _REMOFF)
        rout_copy(step, slot).start()
        return carry

    jax.lax.fori_loop(0, _N_STEPS, body, 0)

    def final_wait(step, slot):
        for c in range(_NCHUNK):
            out_copy(step, slot, c).wait()
        rout_copy(step, slot).wait()

    final_wait(_N_STEPS - 2, 0)
    final_wait(_N_STEPS - 1, 1)


def kernel(probs):
    return pl.pallas_call(
        _pipelined,
        in_specs=[pl.BlockSpec(memory_space=pl.ANY)],
        out_specs=pl.BlockSpec(memory_space=pl.ANY),
        out_shape=jax.ShapeDtypeStruct((_ROWS, _COLS), probs.dtype),
        input_output_aliases={0: 0},
        scratch_shapes=[
            pltpu.VMEM((2, _NCHUNK, _BR, _CW), jnp.float32),
            pltpu.VMEM((2, _NCHUNK, _BR, _CW), jnp.float32),
            pltpu.VMEM((2, _BR, _REMW), jnp.float32),
            pltpu.VMEM((2, _BR, _REMW), jnp.float32),
            pltpu.VMEM((_BR, _CW), jnp.uint32),
            pltpu.SemaphoreType.DMA((2, _NCHUNK)),
            pltpu.SemaphoreType.DMA((2, _NCHUNK)),
            pltpu.SemaphoreType.DMA((2,)),
            pltpu.SemaphoreType.DMA((2,)),
        ],
    )(probs)
